# baseline (speedup 1.0000x reference)
import functools

import jax
import jax.numpy as jnp
from jax import lax
from jax.experimental import pallas as pl
from jax.experimental.pallas import tpu as pltpu
from jax.experimental.pallas import tpu_sc as plsc

_B = 2048
_C = 512
_T = 16
_H = 2
_NOUT = 8
_BT = 32

_NWORK = 32
_B_SC = 0
_B_TC = _B - _B_SC
_SPT = _B_SC // _NWORK
_ROW = _T * _H * 8 * 128
_NCOL = 1024 // 16


def _pool_kernel(x_ref, w_ref, o_ref, tmp0_ref, tmp1_ref, tmp2_ref):
    xb = x_ref[...]
    s = jnp.sum(xb, axis=(1, 2))
    acc = jnp.zeros((_NOUT, xb.shape[0]), jnp.float32)
    for u in range(8):
        acc = acc + jax.lax.dot_general(
            w_ref[u], s[:, u, :],
            (((0,), (1,)), ((), ())),
            preferred_element_type=jnp.float32,
        )
    phase = pl.program_id(0) % 4
    for q, t in enumerate((tmp0_ref, tmp1_ref, tmp2_ref)):
        @pl.when(phase == q)
        def _(t=t):
            t[...] = acc

    @pl.when(phase == 3)
    def _():
        o_ref[...] = jnp.concatenate(
            [tmp0_ref[...], tmp1_ref[...], tmp2_ref[...], acc], axis=1
        )


def _gather16(v, idx):
    return lax.gather(
        v,
        idx[:, None],
        dimension_numbers=lax.GatherDimensionNumbers(
            offset_dims=(), collapsed_slice_dims=(0,), start_index_map=(0,)
        ),
        slice_sizes=(1,),
        mode=lax.GatherScatterMode.PROMISE_IN_BOUNDS,
    )


def _lane_total(v, lanes):
    for sh in (8, 4, 2, 1):
        v = v + _gather16(v, lanes ^ sh)
    return v


def _sc_kernel(x_hbm, w_hbm, o_hbm, xa_v, xb_v, w_v, o_v, sema, semb):
    wid = lax.axis_index("s") * 2 + lax.axis_index("c")
    base = wid * _SPT
    row0 = (_B_TC + base) * _ROW

    pltpu.sync_copy(w_hbm, w_v)
    pltpu.async_copy(x_hbm.at[pl.ds(row0, _ROW)], xa_v, sema)
    if _SPT > 1:
        pltpu.async_copy(x_hbm.at[pl.ds(row0 + _ROW, _ROW)], xb_v, semb)

    def sample_body(i, carry, buf, sem):
        del carry
        pltpu.make_async_copy(x_hbm.at[pl.ds(row0, _ROW)], buf, sem).wait()

        def col_body(col, accs):
            off = col * 16
            s = buf[pl.ds(off, 16)]
            for sp in range(1, 32):
                s = s + buf[pl.ds(off + sp * 1024, 16)]
            return tuple(
                accs[j] + s * w_v[pl.ds(j * 1024 + off, 16)]
                for j in range(5)
            )

        z = jnp.zeros((16,), jnp.float32)
        accs = lax.fori_loop(0, _NCOL, col_body, (z, z, z, z, z))
        lanes = lax.iota(jnp.int32, 16)
        row = w_v[pl.ds(5120, 16)]
        for j in range(5):
            row = jnp.where(lanes == j, row + _lane_total(accs[j], lanes), row)
        o_v[i] = row

        @pl.when(i + 2 < _SPT)
        def _():
            pltpu.async_copy(x_hbm.at[pl.ds(row0 + (i + 2) * _ROW, _ROW)], buf, sem)

        return 0

    def pair_body(g, carry):
        i = 2 * g
        carry = sample_body(i, carry, xa_v, sema)
        carry = sample_body(i + 1, carry, xb_v, semb)
        return carry

    lax.fori_loop(0, _SPT // 2, pair_body, 0)
    pltpu.sync_copy(o_v, o_hbm.at[pl.ds(base, _SPT)])


def kernel(x, start_boundaries, W_loc, b_loc, W_conf, b_conf):
    del start_boundaries
    x6 = x.reshape(_B, 4, 128, _T, _H, 2)
    xv = x6.transpose(0, 3, 4, 1, 5, 2).reshape(_B, _T, _H, 8, 128)

    Wc = jnp.concatenate([W_loc, W_conf], axis=0)
    Wp = jnp.pad(Wc, ((0, _NOUT - 5), (0, 0)))
    wv4 = (Wp.T / (_T * _H * 2)).reshape(4, 1, 128, _NOUT)
    wv = jnp.tile(wv4, (1, 2, 1, 1)).reshape(8, 128, _NOUT)

    out_tc = pl.pallas_call(
        _pool_kernel,
        grid=(_B_TC // _BT,),
        in_specs=[
            pl.BlockSpec((_BT, _T, _H, 8, 128), lambda i: (i, 0, 0, 0, 0)),
            pl.BlockSpec((8, 128, _NOUT), lambda i: (0, 0, 0)),
        ],
        out_specs=pl.BlockSpec((_NOUT, 4 * _BT), lambda i: (0, i // 4)),
        out_shape=jax.ShapeDtypeStruct((_NOUT, _B_TC), jnp.float32),
        scratch_shapes=[
            pltpu.VMEM((_NOUT, _BT), jnp.float32),
            pltpu.VMEM((_NOUT, _BT), jnp.float32),
            pltpu.VMEM((_NOUT, _BT), jnp.float32),
        ],
    )(xv, wv)

    if _B_SC == 0:
        loc = out_tc[:2].T + b_loc
        conf = out_tc[2:5].T + b_conf
        return (loc, conf)

    wg = (Wc / (_T * _H * 2)).reshape(5, 4, 1, 8, 16)
    w_sc = jnp.broadcast_to(wg, (5, 4, 2, 8, 16)).reshape(5 * 1024)
    bias16 = jnp.pad(jnp.concatenate([b_loc, b_conf]), (0, 11))
    w_sc = jnp.concatenate([w_sc, bias16])

    x_flat = xv.reshape(_B * _ROW)

    mesh = plsc.VectorSubcoreMesh(core_axis_name="c", subcore_axis_name="s")
    sc_call = functools.partial(
        pl.kernel,
        mesh=mesh,
        out_type=jax.ShapeDtypeStruct((_B_SC, 16), jnp.float32),
        scratch_types=[
            pltpu.VMEM((_ROW,), jnp.float32),
            pltpu.VMEM((_ROW,), jnp.float32),
            pltpu.VMEM((5136,), jnp.float32),
            pltpu.VMEM((_SPT, 16), jnp.float32),
            pltpu.SemaphoreType.DMA,
            pltpu.SemaphoreType.DMA,
        ],
    )(_sc_kernel)
    out_sc = sc_call(x_flat, w_sc)

    loc = jnp.concatenate([out_tc[:2].T + b_loc, out_sc[:, :2]], axis=0)
    conf = jnp.concatenate([out_tc[2:5].T + b_conf, out_sc[:, 2:5]], axis=0)
    return (loc, conf)


# speedup vs baseline: 1.1126x; 1.1126x over previous
import functools

import jax
import jax.numpy as jnp
from jax import lax
from jax.experimental import pallas as pl
from jax.experimental.pallas import tpu as pltpu
from jax.experimental.pallas import tpu_sc as plsc

_B = 2048
_C = 512
_T = 16
_H = 2
_NOUT = 8
_BT = 64

_NWORK = 32
_B_SC = 0
_B_TC = _B - _B_SC
_SPT = _B_SC // _NWORK
_ROW = _T * _H * 8 * 128
_NCOL = 1024 // 16


def _pool_kernel(x_ref, w_ref, o_ref, tmp_ref):
    xb = x_ref[...]
    s = jnp.sum(xb, axis=(1, 2))
    acc = jnp.zeros((_NOUT, xb.shape[0]), jnp.float32)
    for u in range(8):
        acc = acc + jax.lax.dot_general(
            w_ref[u], s[:, u, :],
            (((0,), (1,)), ((), ())),
            preferred_element_type=jnp.float32,
        )
    is_odd = pl.program_id(0) % 2 == 1

    @pl.when(jnp.logical_not(is_odd))
    def _():
        tmp_ref[...] = acc

    @pl.when(is_odd)
    def _():
        o_ref[...] = jnp.concatenate([tmp_ref[...], acc], axis=1)


def _gather16(v, idx):
    return lax.gather(
        v,
        idx[:, None],
        dimension_numbers=lax.GatherDimensionNumbers(
            offset_dims=(), collapsed_slice_dims=(0,), start_index_map=(0,)
        ),
        slice_sizes=(1,),
        mode=lax.GatherScatterMode.PROMISE_IN_BOUNDS,
    )


def _lane_total(v, lanes):
    for sh in (8, 4, 2, 1):
        v = v + _gather16(v, lanes ^ sh)
    return v


def _sc_kernel(x_hbm, w_hbm, o_hbm, xa_v, xb_v, w_v, o_v, sema, semb):
    wid = lax.axis_index("s") * 2 + lax.axis_index("c")
    base = wid * _SPT
    row0 = (_B_TC + base) * _ROW

    pltpu.sync_copy(w_hbm, w_v)
    pltpu.async_copy(x_hbm.at[pl.ds(row0, _ROW)], xa_v, sema)
    if _SPT > 1:
        pltpu.async_copy(x_hbm.at[pl.ds(row0 + _ROW, _ROW)], xb_v, semb)

    def sample_body(i, carry, buf, sem):
        del carry
        pltpu.make_async_copy(x_hbm.at[pl.ds(row0, _ROW)], buf, sem).wait()

        def col_body(col, accs):
            off = col * 16
            s = buf[pl.ds(off, 16)]
            for sp in range(1, 32):
                s = s + buf[pl.ds(off + sp * 1024, 16)]
            return tuple(
                accs[j] + s * w_v[pl.ds(j * 1024 + off, 16)]
                for j in range(5)
            )

        z = jnp.zeros((16,), jnp.float32)
        accs = lax.fori_loop(0, _NCOL, col_body, (z, z, z, z, z))
        lanes = lax.iota(jnp.int32, 16)
        row = w_v[pl.ds(5120, 16)]
        for j in range(5):
            row = jnp.where(lanes == j, row + _lane_total(accs[j], lanes), row)
        o_v[i] = row

        @pl.when(i + 2 < _SPT)
        def _():
            pltpu.async_copy(x_hbm.at[pl.ds(row0 + (i + 2) * _ROW, _ROW)], buf, sem)

        return 0

    def pair_body(g, carry):
        i = 2 * g
        carry = sample_body(i, carry, xa_v, sema)
        carry = sample_body(i + 1, carry, xb_v, semb)
        return carry

    lax.fori_loop(0, _SPT // 2, pair_body, 0)
    pltpu.sync_copy(o_v, o_hbm.at[pl.ds(base, _SPT)])


def kernel(x, start_boundaries, W_loc, b_loc, W_conf, b_conf):
    del start_boundaries
    x6 = x.reshape(_B, 4, 128, _T, _H, 2)
    xv = x6.transpose(0, 3, 4, 1, 5, 2).reshape(_B, _T, _H, 8, 128)

    Wc = jnp.concatenate([W_loc, W_conf], axis=0)
    Wp = jnp.pad(Wc, ((0, _NOUT - 5), (0, 0)))
    wv4 = (Wp.T / (_T * _H * 2)).reshape(4, 1, 128, _NOUT)
    wv = jnp.tile(wv4, (1, 2, 1, 1)).reshape(8, 128, _NOUT)

    out_tc = pl.pallas_call(
        _pool_kernel,
        grid=(_B_TC // _BT,),
        in_specs=[
            pl.BlockSpec((_BT, _T, _H, 8, 128), lambda i: (i, 0, 0, 0, 0)),
            pl.BlockSpec((8, 128, _NOUT), lambda i: (0, 0, 0)),
        ],
        out_specs=pl.BlockSpec((_NOUT, 2 * _BT), lambda i: (0, i // 2)),
        out_shape=jax.ShapeDtypeStruct((_NOUT, _B_TC), jnp.float32),
        scratch_shapes=[pltpu.VMEM((_NOUT, _BT), jnp.float32)],
    )(xv, wv)

    if _B_SC == 0:
        loc = out_tc[:2].T + b_loc
        conf = out_tc[2:5].T + b_conf
        return (loc, conf)

    wg = (Wc / (_T * _H * 2)).reshape(5, 4, 1, 8, 16)
    w_sc = jnp.broadcast_to(wg, (5, 4, 2, 8, 16)).reshape(5 * 1024)
    bias16 = jnp.pad(jnp.concatenate([b_loc, b_conf]), (0, 11))
    w_sc = jnp.concatenate([w_sc, bias16])

    x_flat = xv.reshape(_B * _ROW)

    mesh = plsc.VectorSubcoreMesh(core_axis_name="c", subcore_axis_name="s")
    sc_call = functools.partial(
        pl.kernel,
        mesh=mesh,
        out_type=jax.ShapeDtypeStruct((_B_SC, 16), jnp.float32),
        scratch_types=[
            pltpu.VMEM((_ROW,), jnp.float32),
            pltpu.VMEM((_ROW,), jnp.float32),
            pltpu.VMEM((5136,), jnp.float32),
            pltpu.VMEM((_SPT, 16), jnp.float32),
            pltpu.SemaphoreType.DMA,
            pltpu.SemaphoreType.DMA,
        ],
    )(_sc_kernel)
    out_sc = sc_call(x_flat, w_sc)

    loc = jnp.concatenate([out_tc[:2].T + b_loc, out_sc[:, :2]], axis=0)
    conf = jnp.concatenate([out_tc[2:5].T + b_conf, out_sc[:, 2:5]], axis=0)
    return (loc, conf)
